# baseline (speedup 1.0000x reference)
import functools

import jax
import jax.numpy as jnp
from jax import lax
from jax.experimental import pallas as pl
from jax.experimental.pallas import tpu as pltpu
from jax.experimental.pallas import tpu_sc as plsc

N = 10000
E = 320000
H = 128
NCORES = 2
NSUB = 16
NTILES = NCORES * NSUB
EPT = E // NTILES
CHUNK = 128
SCHUNK = 160
NCHUNK = SCHUNK // NCORES
NP = 10240
STRIPE = NP // NSUB
ZROWS = 128
DEGW = 16
HH = H // NCORES

_mesh = plsc.VectorSubcoreMesh(core_axis_name="c", subcore_axis_name="s")


def _zero_vmem(buf, rows, width):
    z16 = jnp.zeros((16,), jnp.float32)

    def body(i, carry):
        for k in range(width // 16):
            buf[i, pl.ds(k * 16, 16)] = z16
        return carry

    lax.fori_loop(0, rows, body, 0)


@functools.partial(
    pl.kernel,
    mesh=_mesh,
    compiler_params=pltpu.CompilerParams(use_tc_tiling_on_sc=False),
    out_type=jax.ShapeDtypeStruct((NCORES, NP, DEGW), jnp.float32),
    scratch_types=[
        pltpu.VMEM((NCHUNK, CHUNK), jnp.int32),
        pltpu.VMEM((CHUNK, DEGW), jnp.float32),
        pltpu.VMEM((ZROWS, DEGW), jnp.float32),
        pltpu.VMEM_SHARED((NP, DEGW), jnp.float32),
    ],
)
def _sc_degree(col_hbm, out_hbm, col_v, ones_v, zbuf, acc_sh):
    c = lax.axis_index("c")
    s = lax.axis_index("s")
    pltpu.sync_copy(col_hbm.at[s, pl.ds(c * NCHUNK, NCHUNK)], col_v)
    _zero_vmem(zbuf, ZROWS, DEGW)
    for k in range(STRIPE // ZROWS):
        pltpu.sync_copy(zbuf, acc_sh.at[pl.ds(s * STRIPE + k * ZROWS, ZROWS)])
    o16 = jnp.ones((16,), jnp.float32)

    def fill(i, carry):
        ones_v[i, pl.ds(0, 16)] = o16
        return carry

    lax.fori_loop(0, CHUNK, fill, 0)
    plsc.subcore_barrier()

    def body(j, carry):
        pltpu.sync_copy(ones_v, acc_sh.at[col_v.at[j]], add=True)
        return carry

    lax.fori_loop(0, NCHUNK, body, 0)
    plsc.subcore_barrier()
    pltpu.sync_copy(
        acc_sh.at[pl.ds(s * STRIPE, STRIPE)],
        out_hbm.at[c, pl.ds(s * STRIPE, STRIPE)],
    )


KDEPTH = 2
NGRP = SCHUNK // KDEPTH


@functools.partial(
    pl.kernel,
    mesh=_mesh,
    compiler_params=pltpu.CompilerParams(use_tc_tiling_on_sc=False),
    out_type=jax.ShapeDtypeStruct((NCORES, NP, HH), jnp.float32),
    scratch_types=[
        pltpu.VMEM((SCHUNK, CHUNK), jnp.int32),
        pltpu.VMEM((SCHUNK, CHUNK), jnp.int32),
        pltpu.VMEM((KDEPTH, CHUNK, HH), jnp.float32),
        pltpu.VMEM((KDEPTH, CHUNK, HH), jnp.float32),
        pltpu.VMEM((ZROWS, HH), jnp.float32),
        pltpu.VMEM_SHARED((NP, HH), jnp.float32),
        pltpu.SemaphoreType.DMA,
        pltpu.SemaphoreType.DMA,
        pltpu.SemaphoreType.DMA,
        pltpu.SemaphoreType.DMA,
    ],
)
def _sc_scatter(zs2_hbm, row_hbm, col_hbm, out_hbm, row_v, col_v, bank0, bank1,
                zbuf, acc_sh, gsem0, gsem1, ssem0, ssem1):
    c = lax.axis_index("c")
    s = lax.axis_index("s")
    zs_hbm = zs2_hbm.at[c]
    pltpu.sync_copy(row_hbm.at[s], row_v)
    pltpu.sync_copy(col_hbm.at[s], col_v)
    _zero_vmem(zbuf, ZROWS, HH)
    for k in range(STRIPE // ZROWS):
        pltpu.sync_copy(zbuf, acc_sh.at[pl.ds(s * STRIPE + k * ZROWS, ZROWS)])
    plsc.subcore_barrier()

    def gathers(g, bank, sem, start):
        for b in range(KDEPTH):
            cp = pltpu.make_async_copy(
                zs_hbm.at[row_v.at[g * KDEPTH + b]], bank.at[b], sem)
            cp.start() if start else cp.wait()

    def scatters(g, bank, sem, start):
        for b in range(KDEPTH):
            dst = acc_sh.at[col_v.at[g * KDEPTH + b]]
            if start:
                pltpu.async_copy(bank.at[b], dst, sem, add=True)
            else:
                pltpu.make_async_copy(bank.at[b], dst, sem).wait()

    gathers(0, bank0, gsem0, True)

    def body(i, carry):
        g = 2 * i
        @pl.when(i > 0)
        def _():
            scatters(g - 1, bank1, ssem1, False)

        gathers(g + 1, bank1, gsem1, True)
        gathers(g, bank0, gsem0, False)
        scatters(g, bank0, ssem0, True)
        scatters(g, bank0, ssem0, False)

        @pl.when(i < NGRP // 2 - 1)
        def _():
            gathers(g + 2, bank0, gsem0, True)

        gathers(g + 1, bank1, gsem1, False)
        scatters(g + 1, bank1, ssem1, True)
        return carry

    lax.fori_loop(0, NGRP // 2, body, 0)
    scatters(NGRP - 1, bank1, ssem1, False)
    plsc.subcore_barrier()
    pltpu.sync_copy(
        acc_sh.at[pl.ds(s * STRIPE, STRIPE)],
        out_hbm.at[c, pl.ds(s * STRIPE, STRIPE)],
    )


BN = 2000
NB = N // BN
_DOT = dict(preferred_element_type=jnp.float32, precision=lax.Precision.HIGHEST)


def _write_zs2(zs2_ref, z):
    zs2_ref[0] = z[:, 0:HH]
    zs2_ref[1] = z[:, HH:H]


def _tc_init_body(x_ref, w_ref, b_ref, deg_ref, wc_ref, h_ref, dinv_ref,
                  zs2_ref):
    deg = deg_ref[0, :, 0:1] + deg_ref[1, :, 0:1] + 1.0
    dinv = lax.rsqrt(deg)
    h = jnp.maximum(jnp.dot(x_ref[...], w_ref[...], **_DOT) + b_ref[...], 0.0)
    h_ref[...] = h
    dinv_ref[...] = dinv
    _write_zs2(zs2_ref, jnp.dot(h, wc_ref[...], **_DOT) * dinv)


def _tc_init(x8, w8, b_in, deg2, wc0):
    return pl.pallas_call(
        _tc_init_body,
        grid=(NB,),
        in_specs=[
            pl.BlockSpec((BN, 8), lambda i: (i, 0)),
            pl.BlockSpec((8, H), lambda i: (0, 0)),
            pl.BlockSpec((1, H), lambda i: (0, 0)),
            pl.BlockSpec((NCORES, BN, DEGW), lambda i: (0, i, 0)),
            pl.BlockSpec((H, H), lambda i: (0, 0)),
        ],
        out_specs=[
            pl.BlockSpec((BN, H), lambda i: (i, 0)),
            pl.BlockSpec((BN, 1), lambda i: (i, 0)),
            pl.BlockSpec((NCORES, BN, HH), lambda i: (0, i, 0)),
        ],
        out_shape=[
            jax.ShapeDtypeStruct((N, H), jnp.float32),
            jax.ShapeDtypeStruct((N, 1), jnp.float32),
            jax.ShapeDtypeStruct((NCORES, N, HH), jnp.float32),
        ],
    )(x8, w8, b_in, deg2, wc0)


def _tc_post_a_body(acc_ref, zs_ref, dinv_ref, bc_ref, t_ref, s1_ref, s2_ref,
                    a1, a2):
    i = pl.program_id(0)

    @pl.when(i == 0)
    def _():
        a1[...] = jnp.zeros_like(a1)
        a2[...] = jnp.zeros_like(a2)

    acc = jnp.concatenate([acc_ref[0], acc_ref[1]], axis=1)
    zs = jnp.concatenate([zs_ref[0], zs_ref[1]], axis=1)
    t = dinv_ref[...] * (acc + zs) + bc_ref[...]
    t_ref[...] = t
    a1[...] += jnp.sum(t, axis=0, keepdims=True)
    a2[...] += jnp.sum(t * t, axis=0, keepdims=True)
    s1_ref[...] = a1[...]
    s2_ref[...] = a2[...]


def _tc_post_a(acc2, zs2, dinv, bc_i):
    return pl.pallas_call(
        _tc_post_a_body,
        grid=(NB,),
        in_specs=[
            pl.BlockSpec((NCORES, BN, HH), lambda i: (0, i, 0)),
            pl.BlockSpec((NCORES, BN, HH), lambda i: (0, i, 0)),
            pl.BlockSpec((BN, 1), lambda i: (i, 0)),
            pl.BlockSpec((1, H), lambda i: (0, 0)),
        ],
        out_specs=[
            pl.BlockSpec((BN, H), lambda i: (i, 0)),
            pl.BlockSpec((1, H), lambda i: (0, 0)),
            pl.BlockSpec((1, H), lambda i: (0, 0)),
        ],
        out_shape=[
            jax.ShapeDtypeStruct((N, H), jnp.float32),
            jax.ShapeDtypeStruct((1, H), jnp.float32),
            jax.ShapeDtypeStruct((1, H), jnp.float32),
        ],
        scratch_shapes=[
            pltpu.VMEM((1, H), jnp.float32),
            pltpu.VMEM((1, H), jnp.float32),
        ],
    )(acc2, zs2, dinv, bc_i)


def _norm_relu(t_ref, s1_ref, s2_ref, gw_ref, gb_ref, ga_ref, skip_ref):
    mean = s1_ref[...] * (1.0 / N)
    msq = s2_ref[...] * (1.0 / N)
    ga = ga_ref[...]
    var = msq + (ga * ga - 2.0 * ga) * mean * mean
    inv = lax.rsqrt(var + 1e-5)
    z = (t_ref[...] - ga * mean) * inv * gw_ref[...] + gb_ref[...]
    return jnp.maximum(z, 0.0) + skip_ref[...]


def _tc_post_b_body(t_ref, s1_ref, s2_ref, gw_ref, gb_ref, ga_ref, skip_ref,
                    dinv_ref, wn_ref, h_ref, zs2_ref):
    hn = _norm_relu(t_ref, s1_ref, s2_ref, gw_ref, gb_ref, ga_ref, skip_ref)
    h_ref[...] = hn
    _write_zs2(zs2_ref, jnp.dot(hn, wn_ref[...], **_DOT) * dinv_ref[...])


def _tc_post_b(t, s1, s2, gw_i, gb_i, ga_i, h_skip, dinv, w_next):
    return pl.pallas_call(
        _tc_post_b_body,
        grid=(NB,),
        in_specs=[
            pl.BlockSpec((BN, H), lambda i: (i, 0)),
            pl.BlockSpec((1, H), lambda i: (0, 0)),
            pl.BlockSpec((1, H), lambda i: (0, 0)),
            pl.BlockSpec((1, H), lambda i: (0, 0)),
            pl.BlockSpec((1, H), lambda i: (0, 0)),
            pl.BlockSpec((1, H), lambda i: (0, 0)),
            pl.BlockSpec((BN, H), lambda i: (i, 0)),
            pl.BlockSpec((BN, 1), lambda i: (i, 0)),
            pl.BlockSpec((H, H), lambda i: (0, 0)),
        ],
        out_specs=[
            pl.BlockSpec((BN, H), lambda i: (i, 0)),
            pl.BlockSpec((NCORES, BN, HH), lambda i: (0, i, 0)),
        ],
        out_shape=[
            jax.ShapeDtypeStruct((N, H), jnp.float32),
            jax.ShapeDtypeStruct((NCORES, N, HH), jnp.float32),
        ],
    )(t, s1, s2, gw_i, gb_i, ga_i, h_skip, dinv, w_next)


def _tc_post_last_body(t_ref, s1_ref, s2_ref, gw_ref, gb_ref, ga_ref, skip_ref,
                       h_ref, gm_ref, gx_ref, asum, amax):
    i = pl.program_id(0)

    @pl.when(i == 0)
    def _():
        asum[...] = jnp.zeros_like(asum)
        amax[...] = jnp.full_like(amax, -jnp.inf)

    hn = _norm_relu(t_ref, s1_ref, s2_ref, gw_ref, gb_ref, ga_ref, skip_ref)
    h_ref[...] = hn
    asum[...] += jnp.sum(hn, axis=0, keepdims=True)
    amax[...] = jnp.maximum(amax[...], jnp.max(hn, axis=0, keepdims=True))
    gm_ref[...] = asum[...] * (1.0 / N)
    gx_ref[...] = amax[...]


def _tc_post_last(t, s1, s2, gw_i, gb_i, ga_i, h_skip):
    return pl.pallas_call(
        _tc_post_last_body,
        grid=(NB,),
        in_specs=[
            pl.BlockSpec((BN, H), lambda i: (i, 0)),
            pl.BlockSpec((1, H), lambda i: (0, 0)),
            pl.BlockSpec((1, H), lambda i: (0, 0)),
            pl.BlockSpec((1, H), lambda i: (0, 0)),
            pl.BlockSpec((1, H), lambda i: (0, 0)),
            pl.BlockSpec((1, H), lambda i: (0, 0)),
            pl.BlockSpec((BN, H), lambda i: (i, 0)),
        ],
        out_specs=[
            pl.BlockSpec((BN, H), lambda i: (i, 0)),
            pl.BlockSpec((1, H), lambda i: (0, 0)),
            pl.BlockSpec((1, H), lambda i: (0, 0)),
        ],
        out_shape=[
            jax.ShapeDtypeStruct((N, H), jnp.float32),
            jax.ShapeDtypeStruct((1, H), jnp.float32),
            jax.ShapeDtypeStruct((1, H), jnp.float32),
        ],
        scratch_shapes=[
            pltpu.VMEM((1, H), jnp.float32),
            pltpu.VMEM((1, H), jnp.float32),
        ],
    )(t, s1, s2, gw_i, gb_i, ga_i, h_skip)


def kernel(x, edge_index, W_in, b_in, Wc, bc, gw, gb, ga):
    L = Wc.shape[0]
    pad_e = SCHUNK * CHUNK - E // NSUB
    row16 = jnp.pad(edge_index[0].reshape(NSUB, E // NSUB),
                    ((0, 0), (0, pad_e))).reshape(NSUB, SCHUNK, CHUNK)
    col16 = jnp.pad(edge_index[1].reshape(NSUB, E // NSUB),
                    ((0, 0), (0, pad_e)),
                    constant_values=NP - 1).reshape(NSUB, SCHUNK, CHUNK)
    deg2 = _sc_degree(col16)
    x8 = jnp.pad(x, ((0, 0), (0, 1)))
    w8 = jnp.pad(W_in, ((0, 1), (0, 0)))
    h, dinv, zs = _tc_init(x8, w8, b_in.reshape(1, H), deg2, Wc[0])
    for i in range(L):
        acc2 = _sc_scatter(zs, row16, col16)
        t, s1, s2 = _tc_post_a(acc2, zs, dinv, bc[i].reshape(1, H))
        gwi = gw[i].reshape(1, H)
        gbi = gb[i].reshape(1, H)
        gai = ga[i].reshape(1, H)
        if i < L - 1:
            h, zs = _tc_post_b(t, s1, s2, gwi, gbi, gai, h, dinv, Wc[i + 1])
        else:
            h, gmean, gmax = _tc_post_last(t, s1, s2, gwi, gbi, gai, h)
    graph_emb = jnp.concatenate([gmean[0], gmax[0]])[None, :]
    return (h, graph_emb)


# speedup vs baseline: 1.8665x; 1.8665x over previous
import functools

import jax
import jax.numpy as jnp
from jax import lax
from jax.experimental import pallas as pl
from jax.experimental.pallas import tpu as pltpu
from jax.experimental.pallas import tpu_sc as plsc

N = 10000
E = 320000
H = 128
NCORES = 2
NSUB = 16
NTILES = NCORES * NSUB
EPT = E // NTILES
CHUNK = 125
SCHUNK = E // NSUB // CHUNK
NCHUNK = SCHUNK // NCORES
NP = 10240
STRIPE = NP // NSUB
ZROWS = 128
DEGW = 16
HH = H // NCORES

_mesh = plsc.VectorSubcoreMesh(core_axis_name="c", subcore_axis_name="s")


def _zero_vmem(buf, rows, width):
    z16 = jnp.zeros((16,), jnp.float32)

    def body(i, carry):
        for k in range(width // 16):
            buf[i, pl.ds(k * 16, 16)] = z16
        return carry

    lax.fori_loop(0, rows, body, 0)


@functools.partial(
    pl.kernel,
    mesh=_mesh,
    compiler_params=pltpu.CompilerParams(use_tc_tiling_on_sc=False),
    out_type=jax.ShapeDtypeStruct((NCORES, NP, DEGW), jnp.float32),
    scratch_types=[
        pltpu.VMEM((NCHUNK, CHUNK), jnp.int32),
        pltpu.VMEM((CHUNK, DEGW), jnp.float32),
        pltpu.VMEM((ZROWS, DEGW), jnp.float32),
        pltpu.VMEM_SHARED((NP, DEGW), jnp.float32),
    ],
)
def _sc_degree(col_hbm, out_hbm, col_v, ones_v, zbuf, acc_sh):
    c = lax.axis_index("c")
    s = lax.axis_index("s")
    pltpu.sync_copy(col_hbm.at[s, pl.ds(c * NCHUNK, NCHUNK)], col_v)
    _zero_vmem(zbuf, ZROWS, DEGW)
    for k in range(STRIPE // ZROWS):
        pltpu.sync_copy(zbuf, acc_sh.at[pl.ds(s * STRIPE + k * ZROWS, ZROWS)])
    o16 = jnp.ones((16,), jnp.float32)

    def fill(i, carry):
        ones_v[i, pl.ds(0, 16)] = o16
        return carry

    lax.fori_loop(0, CHUNK, fill, 0)
    plsc.subcore_barrier()

    def body(j, carry):
        pltpu.sync_copy(ones_v, acc_sh.at[col_v.at[j]], add=True)
        return carry

    lax.fori_loop(0, NCHUNK, body, 0)
    plsc.subcore_barrier()
    pltpu.sync_copy(
        acc_sh.at[pl.ds(s * STRIPE, STRIPE)],
        out_hbm.at[c, pl.ds(s * STRIPE, STRIPE)],
    )


KDEPTH = 2
NGRP = SCHUNK // KDEPTH


@functools.partial(
    pl.kernel,
    mesh=_mesh,
    compiler_params=pltpu.CompilerParams(use_tc_tiling_on_sc=False),
    out_type=jax.ShapeDtypeStruct((NCORES, NP, HH), jnp.float32),
    scratch_types=[
        pltpu.VMEM((SCHUNK, CHUNK), jnp.int32),
        pltpu.VMEM((SCHUNK, CHUNK), jnp.int32),
        pltpu.VMEM((KDEPTH, CHUNK, HH), jnp.float32),
        pltpu.VMEM((KDEPTH, CHUNK, HH), jnp.float32),
        pltpu.VMEM((ZROWS, HH), jnp.float32),
        pltpu.VMEM_SHARED((NP, HH), jnp.float32),
        pltpu.SemaphoreType.DMA,
        pltpu.SemaphoreType.DMA,
        pltpu.SemaphoreType.DMA,
        pltpu.SemaphoreType.DMA,
    ],
)
def _sc_scatter(zs2_hbm, row_hbm, col_hbm, out_hbm, row_v, col_v, bank0, bank1,
                zbuf, acc_sh, gsem0, gsem1, ssem0, ssem1):
    c = lax.axis_index("c")
    s = lax.axis_index("s")
    zs_hbm = zs2_hbm.at[c]
    pltpu.sync_copy(row_hbm.at[s], row_v)
    pltpu.sync_copy(col_hbm.at[s], col_v)
    _zero_vmem(zbuf, ZROWS, HH)
    for k in range(STRIPE // ZROWS):
        pltpu.sync_copy(zbuf, acc_sh.at[pl.ds(s * STRIPE + k * ZROWS, ZROWS)])
    plsc.subcore_barrier()

    def gathers(g, bank, sem, start):
        for b in range(KDEPTH):
            cp = pltpu.make_async_copy(
                zs_hbm.at[row_v.at[g * KDEPTH + b]], bank.at[b], sem)
            cp.start() if start else cp.wait()

    def scatters(g, bank, sem, start):
        for b in range(KDEPTH):
            dst = acc_sh.at[col_v.at[g * KDEPTH + b]]
            if start:
                pltpu.async_copy(bank.at[b], dst, sem, add=True)
            else:
                pltpu.make_async_copy(bank.at[b], dst, sem).wait()

    gathers(0, bank0, gsem0, True)

    def body(i, carry):
        g = 2 * i
        @pl.when(i > 0)
        def _():
            scatters(g - 1, bank1, ssem1, False)

        gathers(g + 1, bank1, gsem1, True)
        gathers(g, bank0, gsem0, False)
        scatters(g, bank0, ssem0, True)
        scatters(g, bank0, ssem0, False)

        @pl.when(i < NGRP // 2 - 1)
        def _():
            gathers(g + 2, bank0, gsem0, True)

        gathers(g + 1, bank1, gsem1, False)
        scatters(g + 1, bank1, ssem1, True)
        return carry

    lax.fori_loop(0, NGRP // 2, body, 0)
    scatters(NGRP - 1, bank1, ssem1, False)
    plsc.subcore_barrier()
    pltpu.sync_copy(
        acc_sh.at[pl.ds(s * STRIPE, STRIPE)],
        out_hbm.at[c, pl.ds(s * STRIPE, STRIPE)],
    )


BN = 2000
NB = N // BN
_DOT = dict(preferred_element_type=jnp.float32, precision=lax.Precision.HIGHEST)


def _write_zs2(zs2_ref, z):
    zs2_ref[0] = z[:, 0:HH]
    zs2_ref[1] = z[:, HH:H]


def _tc_init_body(x_ref, w_ref, b_ref, deg_ref, wc_ref, h_ref, dinv_ref,
                  zs2_ref):
    deg = deg_ref[0, :, 0:1] + deg_ref[1, :, 0:1] + 1.0
    dinv = lax.rsqrt(deg)
    h = jnp.maximum(jnp.dot(x_ref[...], w_ref[...], **_DOT) + b_ref[...], 0.0)
    h_ref[...] = h
    dinv_ref[...] = dinv
    _write_zs2(zs2_ref, jnp.dot(h, wc_ref[...], **_DOT) * dinv)


def _tc_init(x8, w8, b_in, deg2, wc0):
    return pl.pallas_call(
        _tc_init_body,
        grid=(NB,),
        in_specs=[
            pl.BlockSpec((BN, 8), lambda i: (i, 0)),
            pl.BlockSpec((8, H), lambda i: (0, 0)),
            pl.BlockSpec((1, H), lambda i: (0, 0)),
            pl.BlockSpec((NCORES, BN, DEGW), lambda i: (0, i, 0)),
            pl.BlockSpec((H, H), lambda i: (0, 0)),
        ],
        out_specs=[
            pl.BlockSpec((BN, H), lambda i: (i, 0)),
            pl.BlockSpec((BN, 1), lambda i: (i, 0)),
            pl.BlockSpec((NCORES, BN, HH), lambda i: (0, i, 0)),
        ],
        out_shape=[
            jax.ShapeDtypeStruct((N, H), jnp.float32),
            jax.ShapeDtypeStruct((N, 1), jnp.float32),
            jax.ShapeDtypeStruct((NCORES, N, HH), jnp.float32),
        ],
    )(x8, w8, b_in, deg2, wc0)


def _tc_post_a_body(acc_ref, zs_ref, dinv_ref, bc_ref, t_ref, s1_ref, s2_ref,
                    a1, a2):
    i = pl.program_id(0)

    @pl.when(i == 0)
    def _():
        a1[...] = jnp.zeros_like(a1)
        a2[...] = jnp.zeros_like(a2)

    acc = jnp.concatenate([acc_ref[0], acc_ref[1]], axis=1)
    zs = jnp.concatenate([zs_ref[0], zs_ref[1]], axis=1)
    t = dinv_ref[...] * (acc + zs) + bc_ref[...]
    t_ref[...] = t
    a1[...] += jnp.sum(t, axis=0, keepdims=True)
    a2[...] += jnp.sum(t * t, axis=0, keepdims=True)
    s1_ref[...] = a1[...]
    s2_ref[...] = a2[...]


def _tc_post_a(acc2, zs2, dinv, bc_i):
    return pl.pallas_call(
        _tc_post_a_body,
        grid=(NB,),
        in_specs=[
            pl.BlockSpec((NCORES, BN, HH), lambda i: (0, i, 0)),
            pl.BlockSpec((NCORES, BN, HH), lambda i: (0, i, 0)),
            pl.BlockSpec((BN, 1), lambda i: (i, 0)),
            pl.BlockSpec((1, H), lambda i: (0, 0)),
        ],
        out_specs=[
            pl.BlockSpec((BN, H), lambda i: (i, 0)),
            pl.BlockSpec((1, H), lambda i: (0, 0)),
            pl.BlockSpec((1, H), lambda i: (0, 0)),
        ],
        out_shape=[
            jax.ShapeDtypeStruct((N, H), jnp.float32),
            jax.ShapeDtypeStruct((1, H), jnp.float32),
            jax.ShapeDtypeStruct((1, H), jnp.float32),
        ],
        scratch_shapes=[
            pltpu.VMEM((1, H), jnp.float32),
            pltpu.VMEM((1, H), jnp.float32),
        ],
    )(acc2, zs2, dinv, bc_i)


def _norm_relu(t_ref, s1_ref, s2_ref, gw_ref, gb_ref, ga_ref, skip_ref):
    mean = s1_ref[...] * (1.0 / N)
    msq = s2_ref[...] * (1.0 / N)
    ga = ga_ref[...]
    var = msq + (ga * ga - 2.0 * ga) * mean * mean
    inv = lax.rsqrt(var + 1e-5)
    z = (t_ref[...] - ga * mean) * inv * gw_ref[...] + gb_ref[...]
    return jnp.maximum(z, 0.0) + skip_ref[...]


def _tc_post_b_body(t_ref, s1_ref, s2_ref, gw_ref, gb_ref, ga_ref, skip_ref,
                    dinv_ref, wn_ref, h_ref, zs2_ref):
    hn = _norm_relu(t_ref, s1_ref, s2_ref, gw_ref, gb_ref, ga_ref, skip_ref)
    h_ref[...] = hn
    _write_zs2(zs2_ref, jnp.dot(hn, wn_ref[...], **_DOT) * dinv_ref[...])


def _tc_post_b(t, s1, s2, gw_i, gb_i, ga_i, h_skip, dinv, w_next):
    return pl.pallas_call(
        _tc_post_b_body,
        grid=(NB,),
        in_specs=[
            pl.BlockSpec((BN, H), lambda i: (i, 0)),
            pl.BlockSpec((1, H), lambda i: (0, 0)),
            pl.BlockSpec((1, H), lambda i: (0, 0)),
            pl.BlockSpec((1, H), lambda i: (0, 0)),
            pl.BlockSpec((1, H), lambda i: (0, 0)),
            pl.BlockSpec((1, H), lambda i: (0, 0)),
            pl.BlockSpec((BN, H), lambda i: (i, 0)),
            pl.BlockSpec((BN, 1), lambda i: (i, 0)),
            pl.BlockSpec((H, H), lambda i: (0, 0)),
        ],
        out_specs=[
            pl.BlockSpec((BN, H), lambda i: (i, 0)),
            pl.BlockSpec((NCORES, BN, HH), lambda i: (0, i, 0)),
        ],
        out_shape=[
            jax.ShapeDtypeStruct((N, H), jnp.float32),
            jax.ShapeDtypeStruct((NCORES, N, HH), jnp.float32),
        ],
    )(t, s1, s2, gw_i, gb_i, ga_i, h_skip, dinv, w_next)


def _tc_post_last_body(t_ref, s1_ref, s2_ref, gw_ref, gb_ref, ga_ref, skip_ref,
                       h_ref, gm_ref, gx_ref, asum, amax):
    i = pl.program_id(0)

    @pl.when(i == 0)
    def _():
        asum[...] = jnp.zeros_like(asum)
        amax[...] = jnp.full_like(amax, -jnp.inf)

    hn = _norm_relu(t_ref, s1_ref, s2_ref, gw_ref, gb_ref, ga_ref, skip_ref)
    h_ref[...] = hn
    asum[...] += jnp.sum(hn, axis=0, keepdims=True)
    amax[...] = jnp.maximum(amax[...], jnp.max(hn, axis=0, keepdims=True))
    gm_ref[...] = asum[...] * (1.0 / N)
    gx_ref[...] = amax[...]


def _tc_post_last(t, s1, s2, gw_i, gb_i, ga_i, h_skip):
    return pl.pallas_call(
        _tc_post_last_body,
        grid=(NB,),
        in_specs=[
            pl.BlockSpec((BN, H), lambda i: (i, 0)),
            pl.BlockSpec((1, H), lambda i: (0, 0)),
            pl.BlockSpec((1, H), lambda i: (0, 0)),
            pl.BlockSpec((1, H), lambda i: (0, 0)),
            pl.BlockSpec((1, H), lambda i: (0, 0)),
            pl.BlockSpec((1, H), lambda i: (0, 0)),
            pl.BlockSpec((BN, H), lambda i: (i, 0)),
        ],
        out_specs=[
            pl.BlockSpec((BN, H), lambda i: (i, 0)),
            pl.BlockSpec((1, H), lambda i: (0, 0)),
            pl.BlockSpec((1, H), lambda i: (0, 0)),
        ],
        out_shape=[
            jax.ShapeDtypeStruct((N, H), jnp.float32),
            jax.ShapeDtypeStruct((1, H), jnp.float32),
            jax.ShapeDtypeStruct((1, H), jnp.float32),
        ],
        scratch_shapes=[
            pltpu.VMEM((1, H), jnp.float32),
            pltpu.VMEM((1, H), jnp.float32),
        ],
    )(t, s1, s2, gw_i, gb_i, ga_i, h_skip)


def kernel(x, edge_index, W_in, b_in, Wc, bc, gw, gb, ga):
    L = Wc.shape[0]
    row16 = edge_index[0].reshape(NSUB, SCHUNK, CHUNK)
    col16 = edge_index[1].reshape(NSUB, SCHUNK, CHUNK)
    deg2 = _sc_degree(col16)
    x8 = jnp.pad(x, ((0, 0), (0, 1)))
    w8 = jnp.pad(W_in, ((0, 1), (0, 0)))
    h, dinv, zs = _tc_init(x8, w8, b_in.reshape(1, H), deg2, Wc[0])
    for i in range(L):
        acc2 = _sc_scatter(zs, row16, col16)
        t, s1, s2 = _tc_post_a(acc2, zs, dinv, bc[i].reshape(1, H))
        gwi = gw[i].reshape(1, H)
        gbi = gb[i].reshape(1, H)
        gai = ga[i].reshape(1, H)
        if i < L - 1:
            h, zs = _tc_post_b(t, s1, s2, gwi, gbi, gai, h, dinv, Wc[i + 1])
        else:
            h, gmean, gmax = _tc_post_last(t, s1, s2, gwi, gbi, gai, h)
    graph_emb = jnp.concatenate([gmean[0], gmax[0]])[None, :]
    return (h, graph_emb)
